# Initial kernel scaffold; baseline (speedup 1.0000x reference)
import jax
import jax.numpy as jnp
from jax.experimental import pallas as pl


def kernel(x, enc_src, enc_dst, proc_src, proc_dst, dec_src, dec_dst, data_attr, hidden_attr, We_src, We_dst, be, W_dp, b_dp, Wm1, bm1, Wm2, bm2, Wp_m0, Wp_h0, bp0, Wp_m1, Wp_h1, bp1, Wd_src, Wd_dst, bd, Wx1, bx1, Wx2, bx2, ln_g, ln_b, Wo, bo):
    raise NotImplementedError("write your pallas kernel here")


# speedup vs baseline: 2.3060x; 2.3060x over previous
import functools

import jax
import jax.numpy as jnp
from jax import lax
from jax.experimental import pallas as pl
from jax.experimental.pallas import tpu as pltpu
from jax.experimental.pallas import tpu_sc as plsc

N_DATA = 50000
N_HID = 10000
C = 128
HID_MLP = 512
OUT = 64

M_ACC = 10240
TRASH_HID = N_HID

RZ = 12544
ACC_R = RZ + 128
OUT_R = 4 * RZ

_f32 = jnp.float32
_i32 = jnp.int32


def _pad_rows(a, rows):
    return jnp.pad(a, ((0, rows - a.shape[0]), (0, 0)))


def _pad_edges(src, dst, e_pad, dst_pad):
    e = src.shape[0]
    src = jnp.concatenate([src.astype(_i32), jnp.zeros((e_pad - e,), _i32)])
    dst = jnp.concatenate([dst.astype(_i32),
                           jnp.full((e_pad - e,), dst_pad, _i32)])
    return src, dst


def _sc_segsum_partials(e_pad):
    EPC = e_pad // 2
    EPT = EPC // 16
    NCH = EPT // 128
    ZCH = M_ACC // 16 // 128

    mesh = plsc.VectorSubcoreMesh(core_axis_name="c", subcore_axis_name="s", num_cores=2, num_subcores=16)

    @functools.partial(
        pl.kernel,
        out_type=jax.ShapeDtypeStruct((2, M_ACC, C), _f32),
        mesh=mesh,
        scratch_types=[
            pltpu.VMEM((128,), _i32),
            pltpu.VMEM((128,), _i32),
            pltpu.VMEM((128, C), _f32),
            pltpu.VMEM_SHARED((M_ACC, C), _f32),
            pltpu.SemaphoreType.DMA,
        ],
    )
    def seg(table, src, dst, out, sidx, didx, rows, acc, sem):
        cid = lax.axis_index("c")
        sid = lax.axis_index("s")

        def zrow(r, carry):
            for t in range(C // 16):
                rows[r, pl.ds(t * 16, 16)] = jnp.zeros((16,), _f32)
            return carry

        lax.fori_loop(0, 128, zrow, 0)
        for k in range(ZCH):
            pltpu.sync_copy(rows, acc.at[pl.ds((sid * ZCH + k) * 128, 128)])
        plsc.subcore_barrier()

        base0 = cid * EPC + sid * EPT

        def step(i, carry):
            b = base0 + i * 128
            pltpu.sync_copy(src.at[pl.ds(b, 128)], sidx)
            pltpu.sync_copy(dst.at[pl.ds(b, 128)], didx)
            pltpu.async_copy(table.at[sidx], rows, sem).wait()
            pltpu.sync_copy(rows, acc.at[didx], add=True)
            return carry

        lax.fori_loop(0, NCH, step, 0)
        plsc.subcore_barrier()
        for k in range(ZCH):
            r0 = (sid * ZCH + k) * 128
            pltpu.sync_copy(acc.at[pl.ds(r0, 128)], out.at[cid, pl.ds(r0, 128)])

    return seg


def _sc_segsum_ranged(e_pad):
    EPT = e_pad // 16
    NCH = EPT // 128
    ZS = ACC_R // 16
    WS = RZ // 16

    mesh = plsc.VectorSubcoreMesh(core_axis_name="c", subcore_axis_name="s", num_cores=2, num_subcores=16)

    @functools.partial(
        pl.kernel,
        out_type=jax.ShapeDtypeStruct((OUT_R, C), _f32),
        mesh=mesh,
        scratch_types=[
            pltpu.VMEM((128,), _i32),
            pltpu.VMEM((128,), _i32),
            pltpu.VMEM((128,), _i32),
            pltpu.VMEM((128, C), _f32),
            pltpu.VMEM_SHARED((ACC_R, C), _f32),
            pltpu.SemaphoreType.DMA,
        ],
    )
    def seg(table, src, dst, out, sidx, didx, gdst, rows, acc, sem):
        cid = lax.axis_index("c")
        sid = lax.axis_index("s")

        def zrow(r, carry):
            for t in range(C // 16):
                rows[r, pl.ds(t * 16, 16)] = jnp.zeros((16,), _f32)
            return carry

        lax.fori_loop(0, 128, zrow, 0)

        for j in range(2):
            lo = (2 * cid + j) * RZ
            off = sid * ZS
            left = ZS
            o = 0
            while left > 0:
                n = min(128, left)
                pltpu.sync_copy(rows.at[pl.ds(0, n)], acc.at[pl.ds(off + o, n)])
                o += n
                left -= n
            plsc.subcore_barrier()

            def step(i, carry):
                b = sid * EPT + i * 128
                pltpu.sync_copy(src.at[pl.ds(b, 128)], sidx)
                pltpu.sync_copy(dst.at[pl.ds(b, 128)], didx)
                for t in range(8):
                    d = didx[pl.ds(t * 16, 16)]
                    rel = d - lo
                    ok = (rel >= 0) & (rel < RZ)
                    gdst[pl.ds(t * 16, 16)] = jnp.where(ok, rel, RZ)
                pltpu.async_copy(table.at[sidx], rows, sem).wait()
                pltpu.sync_copy(rows, acc.at[gdst], add=True)
                return carry

            lax.fori_loop(0, NCH, step, 0)
            plsc.subcore_barrier()
            pltpu.sync_copy(acc.at[pl.ds(sid * WS, WS)],
                            out.at[pl.ds(lo + sid * WS, WS)])
            plsc.subcore_barrier()

    return seg


def _full(shape):
    return pl.BlockSpec(shape, lambda i: tuple(0 for _ in shape))


def _rows(bm, width):
    return pl.BlockSpec((bm, width), lambda i: (i, 0))


def _k1_body(xp, wsrc, wdp, bdp, sp_o, xlat_o):
    x = xp[...]
    sp_o[...] = jnp.dot(x, wsrc[...], preferred_element_type=_f32)
    xlat_o[...] = jax.nn.relu(
        jnp.dot(x, wdp[...], preferred_element_type=_f32) + bdp[...])


def _k2_body(aggp, attrh, wedst, be, wm1, bm1, wm2, bm2, wpm0, wph0, bp0,
             h_o, hm0_o, hw0_o):
    agg = aggp[0] + aggp[1]
    hb = jnp.dot(attrh[...], wedst[...], preferred_element_type=_f32)
    h0 = jax.nn.relu(agg + hb + be[...])
    t = jax.nn.relu(jnp.dot(h0, wm1[...], preferred_element_type=_f32) + bm1[...])
    h = h0 + jnp.dot(t, wm2[...], preferred_element_type=_f32) + bm2[...]
    h_o[...] = h
    hm0_o[...] = jnp.dot(h, wpm0[...], preferred_element_type=_f32)
    hw0_o[...] = jnp.dot(h, wph0[...], preferred_element_type=_f32) + bp0[...]


def _k3_body(h, hw0, m0p, wpm1, wph1, bp1, h1_o, hm1_o, hw1_o):
    h1 = h[...] + jax.nn.relu(m0p[0] + m0p[1] + hw0[...])
    h1_o[...] = h1
    hm1_o[...] = jnp.dot(h1, wpm1[...], preferred_element_type=_f32)
    hw1_o[...] = jnp.dot(h1, wph1[...], preferred_element_type=_f32) + bp1[...]


def _k4_body(h1, hw1, m1p, wdsrc, hd_o):
    h2 = h1[...] + jax.nn.relu(m1p[0] + m1p[1] + hw1[...])
    hd_o[...] = jnp.dot(h2, wdsrc[...], preferred_element_type=_f32)


def _k5_body(dagg, xlat, wddst, bd, wx1, bx1, wx2, bx2, g, b, wo, bo, xlast,
             out_o):
    xd = jax.nn.relu(
        dagg[...] + jnp.dot(xlat[...], wddst[...], preferred_element_type=_f32)
        + bd[...])
    t = jax.nn.relu(jnp.dot(xd, wx1[...], preferred_element_type=_f32) + bx1[...])
    xd = xd + jnp.dot(t, wx2[...], preferred_element_type=_f32) + bx2[...]
    mu = jnp.mean(xd, axis=-1, keepdims=True)
    var = jnp.mean((xd - mu) ** 2, axis=-1, keepdims=True)
    y = (xd - mu) * lax.rsqrt(var + 1e-5) * g[...] + b[...]
    out_o[...] = (jnp.dot(y, wo[...], preferred_element_type=_f32) + bo[...]
                  + xlast[...])


def kernel(x, enc_src, enc_dst, proc_src, proc_dst, dec_src, dec_dst,
           data_attr, hidden_attr, We_src, We_dst, be, W_dp, b_dp, Wm1, bm1,
           Wm2, bm2, Wp_m0, Wp_h0, bp0, Wp_m1, Wp_h1, bp1, Wd_src, Wd_dst, bd,
           Wx1, bx1, Wx2, bx2, ln_g, ln_b, Wo, bo):
    batch, time, ens, grid, nvars = x.shape
    in_dim = time * nvars + data_attr.shape[1]
    kp = 256

    x_flat = jnp.transpose(x, (0, 2, 3, 1, 4)).reshape(grid, time * nvars)
    x_pad = jnp.concatenate(
        [x_flat, data_attr, jnp.zeros((grid, kp - in_dim), _f32)], axis=1)
    wsrc_p = _pad_rows(We_src, kp)
    wdp_p = _pad_rows(W_dp, kp)
    attrh_p = jnp.pad(hidden_attr, ((0, 0), (0, C - hidden_attr.shape[1])))
    wedst_p = _pad_rows(We_dst, C)
    x_last = x[:, -1].reshape(grid, nvars)

    e_enc = ((enc_src.shape[0] + 4095) // 4096) * 4096
    e_proc = ((proc_src.shape[0] + 4095) // 4096) * 4096
    e_dec = ((dec_src.shape[0] + 4095) // 4096) * 4096
    esrc, edst = _pad_edges(enc_src, enc_dst, e_enc, TRASH_HID)
    psrc, pdst = _pad_edges(proc_src, proc_dst, e_proc, TRASH_HID)
    dsrc, ddst = _pad_edges(dec_src, dec_dst, e_dec, 2 * OUT_R)

    r2 = lambda v: v.reshape(1, -1)

    src_proj, x_lat = pl.pallas_call(
        _k1_body,
        grid=(50,),
        in_specs=[_rows(1000, kp), _full((kp, C)), _full((kp, C)),
                  _full((1, C))],
        out_specs=[_rows(1000, C), _rows(1000, C)],
        out_shape=[jax.ShapeDtypeStruct((grid, C), _f32),
                   jax.ShapeDtypeStruct((grid, C), _f32)],
    )(x_pad, wsrc_p, wdp_p, r2(b_dp))

    aggp = _sc_segsum_partials(e_enc)(src_proj, esrc, edst)

    bh = 1000
    h, hm0, hw0 = pl.pallas_call(
        _k2_body,
        grid=(N_HID // bh,),
        in_specs=[pl.BlockSpec((2, bh, C), lambda i: (0, i, 0)),
                  _rows(bh, C), _full((C, C)), _full((1, C)),
                  _full((C, HID_MLP)), _full((1, HID_MLP)),
                  _full((HID_MLP, C)), _full((1, C)),
                  _full((C, C)), _full((C, C)), _full((1, C))],
        out_specs=[_rows(bh, C)] * 3,
        out_shape=[jax.ShapeDtypeStruct((N_HID, C), _f32)] * 3,
    )(aggp, attrh_p, wedst_p, r2(be), Wm1, r2(bm1), Wm2, r2(bm2),
      Wp_m0, Wp_h0, r2(bp0))

    m0p = _sc_segsum_partials(e_proc)(hm0, psrc, pdst)

    h1, hm1, hw1 = pl.pallas_call(
        _k3_body,
        grid=(N_HID // bh,),
        in_specs=[_rows(bh, C), _rows(bh, C),
                  pl.BlockSpec((2, bh, C), lambda i: (0, i, 0)),
                  _full((C, C)), _full((C, C)), _full((1, C))],
        out_specs=[_rows(bh, C)] * 3,
        out_shape=[jax.ShapeDtypeStruct((N_HID, C), _f32)] * 3,
    )(h, hw0, m0p, Wp_m1, Wp_h1, r2(bp1))

    m1p = _sc_segsum_partials(e_proc)(hm1, psrc, pdst)

    hd = pl.pallas_call(
        _k4_body,
        grid=(N_HID // bh,),
        in_specs=[_rows(bh, C), _rows(bh, C),
                  pl.BlockSpec((2, bh, C), lambda i: (0, i, 0)),
                  _full((C, C))],
        out_specs=_rows(bh, C),
        out_shape=jax.ShapeDtypeStruct((N_HID, C), _f32),
    )(h1, hw1, m1p, Wd_src)

    dagg = _sc_segsum_ranged(e_dec)(hd, dsrc, ddst)

    out = pl.pallas_call(
        _k5_body,
        grid=(50,),
        in_specs=[_rows(1000, C), _rows(1000, C), _full((C, C)), _full((1, C)),
                  _full((C, HID_MLP)), _full((1, HID_MLP)),
                  _full((HID_MLP, C)), _full((1, C)),
                  _full((1, C)), _full((1, C)),
                  _full((C, OUT)), _full((1, OUT)), _rows(1000, OUT)],
        out_specs=_rows(1000, OUT),
        out_shape=jax.ShapeDtypeStruct((grid, OUT), _f32),
    )(dagg, x_lat, Wd_dst, r2(bd), Wx1, r2(bx1), Wx2, r2(bx2),
      r2(ln_g), r2(ln_b), Wo, r2(bo), x_last)

    return out.reshape(batch, ens, grid, OUT)


# speedup vs baseline: 2.5504x; 1.1060x over previous
import functools

import jax
import jax.numpy as jnp
from jax import lax
from jax.experimental import pallas as pl
from jax.experimental.pallas import tpu as pltpu
from jax.experimental.pallas import tpu_sc as plsc

N_DATA = 50000
N_HID = 10000
C = 128
HID_MLP = 512
OUT = 64

M_ACC = 10240
TRASH_HID = N_HID

RZ = 6272
NRANGE = 8
ACC_R = RZ + 128
OUT_R = NRANGE * RZ

_f32 = jnp.float32
_i32 = jnp.int32


def _pad_rows(a, rows):
    return jnp.pad(a, ((0, rows - a.shape[0]), (0, 0)))


def _pad_edges(src, dst, e_pad, dst_pad):
    e = src.shape[0]
    src = jnp.concatenate([src.astype(_i32), jnp.zeros((e_pad - e,), _i32)])
    dst = jnp.concatenate([dst.astype(_i32),
                           jnp.full((e_pad - e,), dst_pad, _i32)])
    return src, dst


NB = 2


def _sc_segsum_partials(e_pad):
    EPC = e_pad // 2
    EPT = EPC // 16
    NCH = EPT // 128
    ZCH = M_ACC // 16 // 128
    assert NCH % NB == 0

    mesh = plsc.VectorSubcoreMesh(core_axis_name="c", subcore_axis_name="s", num_cores=2, num_subcores=16)

    @functools.partial(
        pl.kernel,
        out_type=jax.ShapeDtypeStruct((2, M_ACC, C), _f32),
        mesh=mesh,
        scratch_types=[
            pltpu.VMEM((NB, 128), _i32),
            pltpu.VMEM((NB, 128), _i32),
            pltpu.VMEM((NB, 128, C), _f32),
            pltpu.VMEM_SHARED((M_ACC, C), _f32),
        ] + [pltpu.SemaphoreType.DMA] * (2 * NB),
    )
    def seg(table, src, dst, out, sidx, didx, rows, acc, *sems):
        sem_g = sems[:NB]
        sem_s = sems[NB:]
        cid = lax.axis_index("c")
        sid = lax.axis_index("s")

        def zrow(r, carry):
            for t in range(C // 16):
                rows[0, r, pl.ds(t * 16, 16)] = jnp.zeros((16,), _f32)
            return carry

        lax.fori_loop(0, 128, zrow, 0)
        for k in range(ZCH):
            pltpu.sync_copy(rows.at[0],
                            acc.at[pl.ds((sid * ZCH + k) * 128, 128)])
        plsc.subcore_barrier()

        base0 = cid * EPC + sid * EPT

        def gath(b, ch):
            e0 = base0 + ch * 128
            pltpu.sync_copy(src.at[pl.ds(e0, 128)], sidx.at[b])
            pltpu.sync_copy(dst.at[pl.ds(e0, 128)], didx.at[b])
            pltpu.async_copy(table.at[sidx.at[b]], rows.at[b], sem_g[b])

        def scat(b):
            pltpu.async_copy(rows.at[b], acc.at[didx.at[b]], sem_s[b],
                             add=True)

        def wait_g(b):
            pltpu.make_async_copy(table.at[sidx.at[b]], rows.at[b],
                                  sem_g[b]).wait()

        def wait_s(b):
            pltpu.make_async_copy(rows.at[b], acc.at[didx.at[b]],
                                  sem_s[b]).wait()

        def step(k, carry):
            for b in range(NB):
                @pl.when(k > 0)
                def _():
                    wait_s(b)
                gath(b, k * NB + b)
            for b in range(NB):
                wait_g(b)
                scat(b)
            return carry

        lax.fori_loop(0, NCH // NB, step, 0)
        for b in range(NB):
            wait_s(b)
        plsc.subcore_barrier()
        for k in range(ZCH):
            r0 = (sid * ZCH + k) * 128
            pltpu.sync_copy(acc.at[pl.ds(r0, 128)], out.at[cid, pl.ds(r0, 128)])

    return seg


SCCH = 2048
DRAIN_T = 4096
CCAP = DRAIN_T + SCCH + 256
TRASH_PK = RZ << 16


def _sc_segsum_ranged(e_pad):
    EPT = e_pad // 16
    NSC = EPT // SCCH
    RPS = NRANGE // 2
    ZS = ACC_R // 16
    WS = RZ // 16

    mesh = plsc.VectorSubcoreMesh(core_axis_name="c", subcore_axis_name="s", num_cores=2, num_subcores=16)

    @functools.partial(
        pl.kernel,
        out_type=jax.ShapeDtypeStruct((OUT_R, C), _f32),
        mesh=mesh,
        scratch_types=[
            pltpu.VMEM((SCCH,), _i32),
            pltpu.VMEM((SCCH,), _i32),
            pltpu.VMEM((CCAP + 16,), _i32),
            pltpu.VMEM((2, 128), _i32),
            pltpu.VMEM((2, 128), _i32),
            pltpu.VMEM((64, C), _f32),
            pltpu.VMEM((2, 128, C), _f32),
            pltpu.VMEM_SHARED((ACC_R, C), _f32),
        ] + [pltpu.SemaphoreType.DMA] * 4,
    )
    def seg(table, src, dst, out, sbuf, dbuf, cbuf, gsrc, gdst, zbuf,
            rows, acc, *sems):
        sem_g = sems[:2]
        sem_s = sems[2:]
        cid = lax.axis_index("c")
        sid = lax.axis_index("s")

        def zrow(r, carry):
            for t in range(C // 16):
                zbuf[r, pl.ds(t * 16, 16)] = jnp.zeros((16,), _f32)
            return carry

        lax.fori_loop(0, 64, zrow, 0)

        def wait_g(b):
            pltpu.make_async_copy(table.at[gsrc.at[b]], rows.at[b],
                                  sem_g[b]).wait()

        def wait_s(b):
            pltpu.make_async_copy(rows.at[b], acc.at[gdst.at[b]],
                                  sem_s[b]).wait()

        def drain(ptr):
            for t in range(16):
                cbuf[pl.ds(ptr + t * 16, 16)] = jnp.full((16,), RZ << 16, _i32)
            n2 = (ptr + 255) // 256

            def pipe(k, carry):
                for b in range(2):
                    @pl.when(k > 0)
                    def _():
                        wait_s(b)
                    base = (k * 2 + b) * 128
                    for t in range(8):
                        pv = cbuf[pl.ds(base + t * 16, 16)]
                        gsrc[b, pl.ds(t * 16, 16)] = pv & 0xFFFF
                        gdst[b, pl.ds(t * 16, 16)] = pv >> 16
                    pltpu.async_copy(table.at[gsrc.at[b]], rows.at[b], sem_g[b])
                for b in range(2):
                    wait_g(b)
                    pltpu.async_copy(rows.at[b], acc.at[gdst.at[b]], sem_s[b],
                                     add=True)
                return carry

            lax.fori_loop(0, n2, pipe, 0)

            @pl.when(n2 > 0)
            def _():
                wait_s(0)
                wait_s(1)
            return jnp.int32(0)

        for j in range(RPS):
            lo = (RPS * cid + j) * RZ
            off = sid * ZS
            o = 0
            for n in [64] * 6 + [16]:
                pltpu.sync_copy(zbuf.at[pl.ds(0, n)], acc.at[pl.ds(off + o, n)])
                o += n
            plsc.subcore_barrier()

            def scan_chunk(ci, ptr):
                e0 = sid * EPT + ci * SCCH
                pltpu.sync_copy(src.at[pl.ds(e0, SCCH)], sbuf)
                pltpu.sync_copy(dst.at[pl.ds(e0, SCCH)], dbuf)

                def vec(v, p):
                    d = dbuf[pl.ds(v * 16, 16)]
                    s = sbuf[pl.ds(v * 16, 16)]
                    rel = d - lo
                    ok = (rel >= 0) & (rel < RZ)
                    packed = jnp.where(ok, (rel << 16) | s, jnp.int32(TRASH_PK))
                    pos = jnp.where(ok, jnp.int32(1), jnp.int32(0))
                    idx = lax.iota(_i32, 16)
                    for kk in (1, 2, 4, 8):
                        shifted = pos.at[jnp.maximum(idx - kk, 0)].get(
                            mode="promise_in_bounds")
                        pos = pos + jnp.where(idx >= kk, shifted, jnp.int32(0))
                    i1 = idx + 1
                    j = jnp.zeros((16,), _i32)
                    for sz in (8, 4, 2, 1):
                        pv = pos.at[j + (sz - 1)].get(mode="promise_in_bounds")
                        j = j + jnp.where(pv < i1, jnp.int32(sz), jnp.int32(0))
                    comp = packed.at[j].get(mode="promise_in_bounds")
                    cnt = pos[15]
                    comp = jnp.where(idx < cnt, comp, jnp.int32(TRASH_PK))
                    cbuf[pl.ds(p, 16)] = comp
                    return p + cnt

                ptr = lax.fori_loop(0, SCCH // 16, vec, ptr)
                return lax.cond(ptr >= DRAIN_T, drain, lambda p: p, ptr)

            ptr = lax.fori_loop(0, NSC, scan_chunk, jnp.int32(0))
            drain(ptr)

            plsc.subcore_barrier()
            pltpu.sync_copy(acc.at[pl.ds(sid * WS, WS)],
                            out.at[pl.ds(lo + sid * WS, WS)])
            plsc.subcore_barrier()

    return seg


def _full(shape):
    return pl.BlockSpec(shape, lambda i: tuple(0 for _ in shape))


def _rows(bm, width):
    return pl.BlockSpec((bm, width), lambda i: (i, 0))


def _k1_body(xp, wsrc, wdp, bdp, sp_o, xlat_o):
    x = xp[...]
    sp_o[...] = jnp.dot(x, wsrc[...], preferred_element_type=_f32)
    xlat_o[...] = jax.nn.relu(
        jnp.dot(x, wdp[...], preferred_element_type=_f32) + bdp[...])


def _k2_body(aggp, attrh, wedst, be, wm1, bm1, wm2, bm2, wpm0, wph0, bp0,
             h_o, hm0_o, hw0_o):
    agg = aggp[0] + aggp[1]
    hb = jnp.dot(attrh[...], wedst[...], preferred_element_type=_f32)
    h0 = jax.nn.relu(agg + hb + be[...])
    t = jax.nn.relu(jnp.dot(h0, wm1[...], preferred_element_type=_f32) + bm1[...])
    h = h0 + jnp.dot(t, wm2[...], preferred_element_type=_f32) + bm2[...]
    h_o[...] = h
    hm0_o[...] = jnp.dot(h, wpm0[...], preferred_element_type=_f32)
    hw0_o[...] = jnp.dot(h, wph0[...], preferred_element_type=_f32) + bp0[...]


def _k3_body(h, hw0, m0p, wpm1, wph1, bp1, h1_o, hm1_o, hw1_o):
    h1 = h[...] + jax.nn.relu(m0p[0] + m0p[1] + hw0[...])
    h1_o[...] = h1
    hm1_o[...] = jnp.dot(h1, wpm1[...], preferred_element_type=_f32)
    hw1_o[...] = jnp.dot(h1, wph1[...], preferred_element_type=_f32) + bp1[...]


def _k4_body(h1, hw1, m1p, wdsrc, hd_o):
    h2 = h1[...] + jax.nn.relu(m1p[0] + m1p[1] + hw1[...])
    hd_o[...] = jnp.dot(h2, wdsrc[...], preferred_element_type=_f32)


def _k5_body(dagg, xlat, wddst, bd, wx1, bx1, wx2, bx2, g, b, wo, bo, xlast,
             out_o):
    xd = jax.nn.relu(
        dagg[...] + jnp.dot(xlat[...], wddst[...], preferred_element_type=_f32)
        + bd[...])
    t = jax.nn.relu(jnp.dot(xd, wx1[...], preferred_element_type=_f32) + bx1[...])
    xd = xd + jnp.dot(t, wx2[...], preferred_element_type=_f32) + bx2[...]
    mu = jnp.mean(xd, axis=-1, keepdims=True)
    var = jnp.mean((xd - mu) ** 2, axis=-1, keepdims=True)
    y = (xd - mu) * lax.rsqrt(var + 1e-5) * g[...] + b[...]
    out_o[...] = (jnp.dot(y, wo[...], preferred_element_type=_f32) + bo[...]
                  + xlast[...])


def kernel(x, enc_src, enc_dst, proc_src, proc_dst, dec_src, dec_dst,
           data_attr, hidden_attr, We_src, We_dst, be, W_dp, b_dp, Wm1, bm1,
           Wm2, bm2, Wp_m0, Wp_h0, bp0, Wp_m1, Wp_h1, bp1, Wd_src, Wd_dst, bd,
           Wx1, bx1, Wx2, bx2, ln_g, ln_b, Wo, bo):
    batch, time, ens, grid, nvars = x.shape
    in_dim = time * nvars + data_attr.shape[1]
    kp = 256

    x_flat = jnp.transpose(x, (0, 2, 3, 1, 4)).reshape(grid, time * nvars)
    x_pad = jnp.concatenate(
        [x_flat, data_attr, jnp.zeros((grid, kp - in_dim), _f32)], axis=1)
    wsrc_p = _pad_rows(We_src, kp)
    wdp_p = _pad_rows(W_dp, kp)
    attrh_p = jnp.pad(hidden_attr, ((0, 0), (0, C - hidden_attr.shape[1])))
    wedst_p = _pad_rows(We_dst, C)
    x_last = x[:, -1].reshape(grid, nvars)

    e_enc = -(-enc_src.shape[0] // 16384) * 16384
    e_proc = -(-proc_src.shape[0] // 16384) * 16384
    e_dec = -(-dec_src.shape[0] // 32768) * 32768
    esrc, edst = _pad_edges(enc_src, enc_dst, e_enc, TRASH_HID)
    psrc, pdst = _pad_edges(proc_src, proc_dst, e_proc, TRASH_HID)
    dsrc, ddst = _pad_edges(dec_src, dec_dst, e_dec, 2 * OUT_R)

    r2 = lambda v: v.reshape(1, -1)

    src_proj, x_lat = pl.pallas_call(
        _k1_body,
        grid=(50,),
        in_specs=[_rows(1000, kp), _full((kp, C)), _full((kp, C)),
                  _full((1, C))],
        out_specs=[_rows(1000, C), _rows(1000, C)],
        out_shape=[jax.ShapeDtypeStruct((grid, C), _f32),
                   jax.ShapeDtypeStruct((grid, C), _f32)],
    )(x_pad, wsrc_p, wdp_p, r2(b_dp))

    aggp = _sc_segsum_partials(e_enc)(src_proj, esrc, edst)

    bh = 1000
    h, hm0, hw0 = pl.pallas_call(
        _k2_body,
        grid=(N_HID // bh,),
        in_specs=[pl.BlockSpec((2, bh, C), lambda i: (0, i, 0)),
                  _rows(bh, C), _full((C, C)), _full((1, C)),
                  _full((C, HID_MLP)), _full((1, HID_MLP)),
                  _full((HID_MLP, C)), _full((1, C)),
                  _full((C, C)), _full((C, C)), _full((1, C))],
        out_specs=[_rows(bh, C)] * 3,
        out_shape=[jax.ShapeDtypeStruct((N_HID, C), _f32)] * 3,
    )(aggp, attrh_p, wedst_p, r2(be), Wm1, r2(bm1), Wm2, r2(bm2),
      Wp_m0, Wp_h0, r2(bp0))

    m0p = _sc_segsum_partials(e_proc)(hm0, psrc, pdst)

    h1, hm1, hw1 = pl.pallas_call(
        _k3_body,
        grid=(N_HID // bh,),
        in_specs=[_rows(bh, C), _rows(bh, C),
                  pl.BlockSpec((2, bh, C), lambda i: (0, i, 0)),
                  _full((C, C)), _full((C, C)), _full((1, C))],
        out_specs=[_rows(bh, C)] * 3,
        out_shape=[jax.ShapeDtypeStruct((N_HID, C), _f32)] * 3,
    )(h, hw0, m0p, Wp_m1, Wp_h1, r2(bp1))

    m1p = _sc_segsum_partials(e_proc)(hm1, psrc, pdst)

    hd = pl.pallas_call(
        _k4_body,
        grid=(N_HID // bh,),
        in_specs=[_rows(bh, C), _rows(bh, C),
                  pl.BlockSpec((2, bh, C), lambda i: (0, i, 0)),
                  _full((C, C))],
        out_specs=_rows(bh, C),
        out_shape=jax.ShapeDtypeStruct((N_HID, C), _f32),
    )(h1, hw1, m1p, Wd_src)

    dagg = _sc_segsum_ranged(e_dec)(hd, dsrc, ddst)

    out = pl.pallas_call(
        _k5_body,
        grid=(50,),
        in_specs=[_rows(1000, C), _rows(1000, C), _full((C, C)), _full((1, C)),
                  _full((C, HID_MLP)), _full((1, HID_MLP)),
                  _full((HID_MLP, C)), _full((1, C)),
                  _full((1, C)), _full((1, C)),
                  _full((C, OUT)), _full((1, OUT)), _rows(1000, OUT)],
        out_specs=_rows(1000, OUT),
        out_shape=jax.ShapeDtypeStruct((grid, OUT), _f32),
    )(dagg, x_lat, Wd_dst, r2(bd), Wx1, r2(bx1), Wx2, r2(bx2),
      r2(ln_g), r2(ln_b), Wo, r2(bo), x_last)

    return out.reshape(batch, ens, grid, OUT)
